# baseline (speedup 1.0000x reference)
import functools

import jax
import jax.numpy as jnp
from jax import lax
from jax.experimental import pallas as pl
from jax.experimental.pallas import tpu as pltpu
from jax.experimental.pallas import tpu_sc as plsc

VOCAB = 100000
D = 64
B = 16384
S = 50
L = 16
NC = 2
NS = 16
NW = NC * NS
RPT = B // NW
CH = 16
NCHUNK = RPT // CH
UNROLL = 50


def _dyngather(v, idx):
    return lax.gather(
        v,
        idx[:, None],
        lax.GatherDimensionNumbers(
            offset_dims=(), collapsed_slice_dims=(0,), start_index_map=(0,)
        ),
        slice_sizes=(1,),
        mode=lax.GatherScatterMode.PROMISE_IN_BOUNDS,
    )


_mesh = plsc.VectorSubcoreMesh(
    core_axis_name="c", subcore_axis_name="s", num_cores=NC, num_subcores=NS
)


@functools.partial(
    pl.kernel,
    out_type=jax.ShapeDtypeStruct((B, D), jnp.float32),
    mesh=_mesh,
    scratch_types=[
        pltpu.VMEM((2, CH, S), jnp.int32),
        pltpu.VMEM((2, CH * S, D), jnp.bfloat16),
        pltpu.VMEM((2, CH, D), jnp.float32),
        pltpu.SemaphoreType.DMA,
        pltpu.SemaphoreType.DMA,
        pltpu.SemaphoreType.DMA,
        pltpu.SemaphoreType.DMA,
    ],
    compiler_params=pltpu.CompilerParams(
        use_tc_tiling_on_sc=False, needs_layout_passes=False
    ),
)
def _embed_pool(idx_hbm, table_hbm, out_hbm, idx_v, rows_v, out_v,
                sem0, sem1, osem0, osem1):
    sems = (sem0, sem1)
    osems = (osem0, osem1)
    wid = lax.axis_index("s") * NC + lax.axis_index("c")
    base = wid * RPT

    def fire(b, ci):
        row0 = base + ci * CH
        pltpu.sync_copy(idx_hbm.at[pl.ds(row0, CH)], idx_v.at[b])
        for r in range(CH):
            pltpu.async_copy(
                table_hbm.at[idx_v.at[b, r]],
                rows_v.at[b, pl.ds(r * S, S)],
                sems[b],
            )

    def drain(b):
        for r in range(CH):
            pltpu.make_async_copy(
                table_hbm.at[idx_v.at[b, r]],
                rows_v.at[b, pl.ds(r * S, S)],
                sems[b],
            ).wait()

    def process(b, ci, first):
        row0 = base + ci * CH

        @pl.when(jnp.logical_not(first))
        def _():
            pltpu.make_async_copy(
                out_v.at[b], out_hbm.at[pl.ds(base, CH)], osems[b]
            ).wait()

        lanes = lax.iota(jnp.int32, L)
        for r in range(CH):
            zero = jnp.zeros((L,), jnp.float32)

            @plsc.parallel_loop(0, S, unroll=UNROLL,
                                carry=(zero, zero, zero, zero))
            def accs(j, acc):
                rb = r * S + j
                out = []
                for h in range(2):
                    pair = rows_v[b, rb, pl.ds(h * 2 * L, 2 * L)]
                    lo, hi = plsc.unpack(
                        pair, format=plsc.PackFormat.INTERLEAVED,
                        preferred_element_type=jnp.float32)
                    out.append(acc[2 * h] + lo)
                    out.append(acc[2 * h + 1] + hi)
                return tuple(out)

            cnt = jnp.zeros((L,), jnp.int32)
            for k in range(3):
                v = idx_v[b, r, pl.ds(k * L, L)]
                cnt = cnt + jnp.where(v != 0, 1, 0)
            v3 = idx_v[b, r, pl.ds(S - L, L)]
            cnt = cnt + jnp.where(
                (v3 != 0) & (lanes >= 2 * L - (S - 2 * L)), 1, 0
            )
            cntf = cnt.astype(jnp.float32)
            for sh in (8, 4, 2, 1):
                cntf = cntf + _dyngather(cntf, lanes ^ sh)
            inv = 1.0 / jnp.maximum(cntf, 1e-9)
            for k in range(4):
                out_v[b, r, pl.ds(k * L, L)] = accs[k] * inv
        pltpu.async_copy(out_v.at[b], out_hbm.at[pl.ds(row0, CH)], osems[b])

    fire(0, 0)

    def pair_body(g, carry):
        fire(1, 2 * g + 1)
        drain(0)
        process(0, 2 * g, g == 0)

        @pl.when(g < NCHUNK // 2 - 1)
        def _():
            fire(0, 2 * g + 2)

        drain(1)
        process(1, 2 * g + 1, g == 0)
        return carry

    lax.fori_loop(0, NCHUNK // 2, pair_body, 0)
    for b in range(2):
        pltpu.make_async_copy(
            out_v.at[b], out_hbm.at[pl.ds(base, CH)], osems[b]
        ).wait()


def kernel(text_input_per_row, embedding):
    idx = text_input_per_row.astype(jnp.int32)
    tbl = (
        embedding.astype(jnp.bfloat16)
        .reshape(VOCAB, 2, 2, L)
        .transpose(0, 1, 3, 2)
        .reshape(VOCAB, D)
    )
    return _embed_pool(idx, tbl)


# speedup vs baseline: 1.6588x; 1.6588x over previous
import functools

import jax
import jax.numpy as jnp
from jax import lax
from jax.experimental import pallas as pl
from jax.experimental.pallas import tpu as pltpu
from jax.experimental.pallas import tpu_sc as plsc

VOCAB = 100000
D = 64
B = 16384
S = 50
L = 16
NC = 2
NS = 16
NW = NC * NS
RPT = B // NW
CH = 8
NCHUNK = RPT // CH
UNROLL = 50


def _dyngather(v, idx):
    return lax.gather(
        v,
        idx[:, None],
        lax.GatherDimensionNumbers(
            offset_dims=(), collapsed_slice_dims=(0,), start_index_map=(0,)
        ),
        slice_sizes=(1,),
        mode=lax.GatherScatterMode.PROMISE_IN_BOUNDS,
    )


_mesh = plsc.VectorSubcoreMesh(
    core_axis_name="c", subcore_axis_name="s", num_cores=NC, num_subcores=NS
)


@functools.partial(
    pl.kernel,
    out_type=jax.ShapeDtypeStruct((B, D), jnp.float32),
    mesh=_mesh,
    scratch_types=[
        pltpu.VMEM((2, CH, S), jnp.int32),
        pltpu.VMEM((2, CH * S, D), jnp.float32),
        pltpu.VMEM((2, CH, D), jnp.float32),
        pltpu.SemaphoreType.DMA,
        pltpu.SemaphoreType.DMA,
        pltpu.SemaphoreType.DMA,
        pltpu.SemaphoreType.DMA,
    ],
    compiler_params=pltpu.CompilerParams(use_tc_tiling_on_sc=False),
)
def _embed_pool(idx_hbm, table_hbm, out_hbm, idx_v, rows_v, out_v,
                sem0, sem1, osem0, osem1):
    sems = (sem0, sem1)
    osems = (osem0, osem1)
    wid = lax.axis_index("s") * NC + lax.axis_index("c")
    base = wid * RPT

    def fire(b, ci):
        row0 = base + ci * CH
        pltpu.sync_copy(idx_hbm.at[pl.ds(row0, CH)], idx_v.at[b])
        for r in range(CH):
            pltpu.async_copy(
                table_hbm.at[idx_v.at[b, r]],
                rows_v.at[b, pl.ds(r * S, S)],
                sems[b],
            )

    def drain(b):
        for r in range(CH):
            pltpu.make_async_copy(
                table_hbm.at[idx_v.at[b, r]],
                rows_v.at[b, pl.ds(r * S, S)],
                sems[b],
            ).wait()

    def process(b, ci, first):
        row0 = base + ci * CH

        @pl.when(jnp.logical_not(first))
        def _():
            pltpu.make_async_copy(
                out_v.at[b], out_hbm.at[pl.ds(base, CH)], osems[b]
            ).wait()

        lanes = lax.iota(jnp.int32, L)
        for r2 in range(CH // 2):
            zero = jnp.zeros((L,), jnp.float32)

            @plsc.parallel_loop(0, S, unroll=UNROLL, carry=(zero,) * 8)
            def acc8(j, acc):
                out = []
                for p in range(2):
                    rb = (2 * r2 + p) * S + j
                    for k in range(4):
                        out.append(
                            acc[4 * p + k] + rows_v[b, rb, pl.ds(k * L, L)]
                        )
                return tuple(out)

            for p in range(2):
                r = 2 * r2 + p
                cnt = jnp.zeros((L,), jnp.int32)
                for k in range(3):
                    v = idx_v[b, r, pl.ds(k * L, L)]
                    cnt = cnt + jnp.where(v != 0, 1, 0)
                v3 = idx_v[b, r, pl.ds(S - L, L)]
                cnt = cnt + jnp.where(
                    (v3 != 0) & (lanes >= 2 * L - (S - 2 * L)), 1, 0
                )
                cntf = cnt.astype(jnp.float32)
                for sh in (8, 4, 2, 1):
                    cntf = cntf + _dyngather(cntf, lanes ^ sh)
                inv = 1.0 / jnp.maximum(cntf, 1e-9)
                for k in range(4):
                    out_v[b, r, pl.ds(k * L, L)] = acc8[4 * p + k] * inv
        pltpu.async_copy(out_v.at[b], out_hbm.at[pl.ds(row0, CH)], osems[b])

    fire(0, 0)

    def pair_body(g, carry):
        fire(1, 2 * g + 1)
        drain(0)
        process(0, 2 * g, g == 0)

        @pl.when(g < NCHUNK // 2 - 1)
        def _():
            fire(0, 2 * g + 2)

        drain(1)
        process(1, 2 * g + 1, g == 0)
        return carry

    lax.fori_loop(0, NCHUNK // 2, pair_body, 0)
    for b in range(2):
        pltpu.make_async_copy(
            out_v.at[b], out_hbm.at[pl.ds(base, CH)], osems[b]
        ).wait()


def kernel(text_input_per_row, embedding):
    idx = text_input_per_row.astype(jnp.int32)
    return _embed_pool(idx, embedding)


# speedup vs baseline: 1.6714x; 1.0076x over previous
import functools

import jax
import jax.numpy as jnp
from jax import lax
from jax.experimental import pallas as pl
from jax.experimental.pallas import tpu as pltpu
from jax.experimental.pallas import tpu_sc as plsc

VOCAB = 100000
D = 64
B = 16384
S = 50
L = 16
NC = 2
NS = 16
NW = NC * NS
RPT = B // NW
CH = 8
NCHUNK = RPT // CH
UNROLL = 50


def _dyngather(v, idx):
    return lax.gather(
        v,
        idx[:, None],
        lax.GatherDimensionNumbers(
            offset_dims=(), collapsed_slice_dims=(0,), start_index_map=(0,)
        ),
        slice_sizes=(1,),
        mode=lax.GatherScatterMode.PROMISE_IN_BOUNDS,
    )


_mesh = plsc.VectorSubcoreMesh(
    core_axis_name="c", subcore_axis_name="s", num_cores=NC, num_subcores=NS
)


@functools.partial(
    pl.kernel,
    out_type=jax.ShapeDtypeStruct((B, D), jnp.float32),
    mesh=_mesh,
    scratch_types=[
        pltpu.VMEM((2, CH, S), jnp.int32),
        pltpu.VMEM((2, CH * S, D), jnp.float32),
        pltpu.VMEM((2, CH, D), jnp.float32),
        pltpu.SemaphoreType.DMA,
        pltpu.SemaphoreType.DMA,
        pltpu.SemaphoreType.DMA,
        pltpu.SemaphoreType.DMA,
    ],
    compiler_params=pltpu.CompilerParams(use_tc_tiling_on_sc=False),
)
def _embed_pool(idx_hbm, table_hbm, out_hbm, idx_v, rows_v, out_v,
                sem0, sem1, osem0, osem1):
    sems = (sem0, sem1)
    osems = (osem0, osem1)
    wid = lax.axis_index("s") * NC + lax.axis_index("c")
    base = wid * RPT

    def fire(b, ci):
        row0 = base + ci * CH
        pltpu.sync_copy(idx_hbm.at[pl.ds(row0, CH)], idx_v.at[b])
        for r in range(CH):
            pltpu.async_copy(
                table_hbm.at[idx_v.at[b, r]],
                rows_v.at[b, pl.ds(r * S, S)],
                sems[b],
            )

    def drain(b):
        for r in range(CH):
            pltpu.make_async_copy(
                table_hbm.at[idx_v.at[b, r]],
                rows_v.at[b, pl.ds(r * S, S)],
                sems[b],
            ).wait()

    def process(b, ci, first):
        row0 = base + ci * CH

        @pl.when(jnp.logical_not(first))
        def _():
            pltpu.make_async_copy(
                out_v.at[b], out_hbm.at[pl.ds(base, CH)], osems[b]
            ).wait()

        lanes = lax.iota(jnp.int32, L)
        for r2 in range(CH // 4):
            zero = jnp.zeros((L,), jnp.float32)

            @plsc.parallel_loop(0, S, unroll=UNROLL, carry=(zero,) * 16)
            def acc8(j, acc):
                out = []
                for p in range(4):
                    rb = (4 * r2 + p) * S + j
                    for k in range(4):
                        out.append(
                            acc[4 * p + k] + rows_v[b, rb, pl.ds(k * L, L)]
                        )
                return tuple(out)

            for p in range(4):
                r = 4 * r2 + p
                cnt = jnp.zeros((L,), jnp.int32)
                for k in range(3):
                    v = idx_v[b, r, pl.ds(k * L, L)]
                    cnt = cnt + jnp.where(v != 0, 1, 0)
                v3 = idx_v[b, r, pl.ds(S - L, L)]
                cnt = cnt + jnp.where(
                    (v3 != 0) & (lanes >= 2 * L - (S - 2 * L)), 1, 0
                )
                cntf = cnt.astype(jnp.float32)
                for sh in (8, 4, 2, 1):
                    cntf = cntf + _dyngather(cntf, lanes ^ sh)
                inv = 1.0 / jnp.maximum(cntf, 1e-9)
                for k in range(4):
                    out_v[b, r, pl.ds(k * L, L)] = acc8[4 * p + k] * inv
        pltpu.async_copy(out_v.at[b], out_hbm.at[pl.ds(row0, CH)], osems[b])

    fire(0, 0)

    def pair_body(g, carry):
        fire(1, 2 * g + 1)
        drain(0)
        process(0, 2 * g, g == 0)

        @pl.when(g < NCHUNK // 2 - 1)
        def _():
            fire(0, 2 * g + 2)

        drain(1)
        process(1, 2 * g + 1, g == 0)
        return carry

    lax.fori_loop(0, NCHUNK // 2, pair_body, 0)
    for b in range(2):
        pltpu.make_async_copy(
            out_v.at[b], out_hbm.at[pl.ds(base, CH)], osems[b]
        ).wait()


def kernel(text_input_per_row, embedding):
    idx = text_input_per_row.astype(jnp.int32)
    return _embed_pool(idx, embedding)
